# Initial kernel scaffold; baseline (speedup 1.0000x reference)
import jax
import jax.numpy as jnp
from jax.experimental import pallas as pl


def kernel(h, edge_index, W_self, b_self, W_neigh, b_neigh, ln_g, ln_b):
    raise NotImplementedError("write your pallas kernel here")


# speedup vs baseline: 4.3041x; 4.3041x over previous
import jax
import jax.numpy as jnp
from jax import lax
from jax.experimental import pallas as pl
from jax.experimental.pallas import tpu as pltpu
from jax.experimental.pallas import tpu_sc as plsc

N = 10000
E = 320000
D = 128
DA = D + 16
NC = 2
NS = 16
NW = NC * NS
BLK = 128
ITERS = -(-E // (NW * BLK))
E_PAD = NW * BLK * ITERS
NPAD = 10112
RPT = NPAD // NS


def _sc_body(ha_hbm, src_hbm, dst_hbm, zrow_hbm, out, sh, idx_s, idx_d, rows, zv, sem):
    c = lax.axis_index("c")
    s = lax.axis_index("s")
    wid = s * NC + c

    pltpu.sync_copy(zrow_hbm, zv)
    r0 = s * RPT

    def zb(i, carry):
        pltpu.sync_copy(zv, sh.at[pl.ds(r0 + i * 8, 8)])
        return carry
    lax.fori_loop(0, RPT // 8, zb, 0)
    plsc.subcore_barrier()

    @pl.loop(0, ITERS)
    def _blocks(i):
        base = (wid + NW * i) * BLK
        pltpu.sync_copy(src_hbm.at[pl.ds(base, BLK)], idx_s)
        pltpu.sync_copy(dst_hbm.at[pl.ds(base, BLK)], idx_d)
        pltpu.async_copy(ha_hbm.at[idx_s], rows, sem).wait()
        pltpu.sync_copy(rows, sh.at[idx_d], add=True)

    plsc.subcore_barrier()
    pltpu.sync_copy(sh.at[pl.ds(r0, RPT)], out.at[c].at[pl.ds(r0, RPT)])


def _sc_aggregate(h_aug, src, dst, zrow):
    mesh = plsc.VectorSubcoreMesh(core_axis_name="c", subcore_axis_name="s",
                                  num_cores=NC, num_subcores=NS)
    return pl.kernel(
        _sc_body,
        out_type=jax.ShapeDtypeStruct((NC, NPAD, DA), jnp.float32),
        mesh=mesh,
        compiler_params=pltpu.CompilerParams(use_tc_tiling_on_sc=False),
        scratch_types=[
            pltpu.VMEM_SHARED((NPAD, DA), jnp.float32),
            pltpu.VMEM((BLK,), jnp.int32),
            pltpu.VMEM((BLK,), jnp.int32),
            pltpu.VMEM((BLK, DA), jnp.float32),
            pltpu.VMEM((8, DA), jnp.float32),
            pltpu.SemaphoreType.DMA,
        ],
    )(h_aug, src, dst, zrow)


ROWS_TC = 400


def _tc_body(h, a0, a1, ws, wn, bs, bn, g, b, o):
    deg = jnp.maximum(a0[:, D:D + 1] + a1[:, D:D + 1], 1.0)
    neigh = (a0[:, :D] + a1[:, :D]) / deg
    dn = (((1,), (1,)), ((), ()))
    z = (lax.dot_general(h[...], ws[...], dn, preferred_element_type=jnp.float32)
         + bs[...]
         + lax.dot_general(neigh, wn[...], dn, preferred_element_type=jnp.float32)
         + bn[...])
    z = z * jax.nn.sigmoid(z)
    mu = jnp.mean(z, axis=-1, keepdims=True)
    r = z - mu
    var = jnp.mean(r * r, axis=-1, keepdims=True)
    o[...] = r * lax.rsqrt(var + 1e-5) * g[...] + b[...]


def _tc_dense(h, a0, a1, W_self, W_neigh, b_self, b_neigh, ln_g, ln_b):
    grid = (N // ROWS_TC,)
    row_spec = pl.BlockSpec((ROWS_TC, D), lambda i: (i, 0))
    acc_spec = pl.BlockSpec((ROWS_TC, DA), lambda i: (i, 0))
    w_spec = pl.BlockSpec((D, D), lambda i: (0, 0))
    v_spec = pl.BlockSpec((1, D), lambda i: (0, 0))
    return pl.pallas_call(
        _tc_body,
        grid=grid,
        in_specs=[row_spec, acc_spec, acc_spec,
                  w_spec, w_spec, v_spec, v_spec, v_spec, v_spec],
        out_specs=row_spec,
        out_shape=jax.ShapeDtypeStruct((N, D), jnp.float32),
    )(h, a0, a1, W_self, W_neigh,
      b_self.reshape(1, D), b_neigh.reshape(1, D),
      ln_g.reshape(1, D), ln_b.reshape(1, D))


def kernel(h, edge_index, W_self, b_self, W_neigh, b_neigh, ln_g, ln_b):
    pad = E_PAD - E
    src = jnp.concatenate([edge_index[0], jnp.zeros((pad,), jnp.int32)])
    dst = jnp.concatenate([edge_index[1], jnp.full((pad,), N, jnp.int32)])
    h_aug = jnp.concatenate(
        [h, jnp.ones((N, 1), jnp.float32), jnp.zeros((N, DA - D - 1), jnp.float32)],
        axis=1)
    zrow = jnp.zeros((8, DA), jnp.float32)
    acc = _sc_aggregate(h_aug, src, dst, zrow)
    return _tc_dense(h, acc[0, :N], acc[1, :N],
                     W_self, W_neigh, b_self, b_neigh, ln_g, ln_b)
